# baseline (speedup 1.0000x reference)
import functools

import jax
import jax.numpy as jnp
from jax import lax
from jax.experimental import pallas as pl
from jax.experimental.pallas import tpu as pltpu
from jax.experimental.pallas import tpu_sc as plsc

N = 10000
E = 320000
DF = 128
DE = 16
L = 64
OUT = 64
G = 64

NC = 2
NS = 16
NW = NC * NS

IDXW = 125
CHUNK = 250
SPT = E // NS
SROUNDS = SPT // CHUNK
EPW = E // NW
GROUNDS = EPW // CHUNK
NSTR = CHUNK // IDXW

ROWS_MAIN = 624
TAIL_BASE = NS * ROWS_MAIN
TAIL = N - TAIL_BASE
HALF = N // NC
WOUT = 312
WTAIL_BASE = NS * WOUT
WTAIL = HALF - WTAIL_BASE

_mesh = functools.partial(
    plsc.VectorSubcoreMesh, core_axis_name="c", subcore_axis_name="s",
    num_cores=NC, num_subcores=NS)

_sc_params = pltpu.CompilerParams(use_tc_tiling_on_sc=False)


def _zero_table(ztab, table, sid):
    pltpu.sync_copy(ztab.at[pl.ds(sid * ROWS_MAIN, ROWS_MAIN)],
                    table.at[pl.ds(sid * ROWS_MAIN, ROWS_MAIN)])

    @pl.when(sid == 0)
    def _():
        pltpu.sync_copy(ztab.at[pl.ds(TAIL_BASE, TAIL)],
                        table.at[pl.ds(TAIL_BASE, TAIL)])


def _scatter_phase(msgs, table, didx, bufa, bufb, lda, ldb, sta, stb,
                   ebase, nrounds):
    def _refill(r, buf, sem):
        pltpu.async_copy(msgs.at[pl.ds(ebase + r * CHUNK, CHUNK)], buf, sem)

    def _wait_refill(r, buf, sem):
        pltpu.make_async_copy(
            msgs.at[pl.ds(ebase + r * CHUNK, CHUNK)], buf, sem).wait()

    def _scatters(r, buf, sem):
        for k in range(NSTR):
            pltpu.async_copy(buf.at[pl.ds(k * IDXW, IDXW)],
                             table.at[didx.at[r * NSTR + k]], sem, add=True)

    def _wait_scatters(r, buf, sem):
        for k in range(NSTR):
            pltpu.make_async_copy(buf.at[pl.ds(k * IDXW, IDXW)],
                                  table.at[didx.at[r * NSTR + k]], sem).wait()

    _refill(0, bufa, lda)

    def body(q, carry):
        ra = 2 * q
        rb = 2 * q + 1

        @pl.when(q >= 1)
        def _():
            _wait_scatters(rb - 2, bufb, stb)

        _refill(rb, bufb, ldb)
        _wait_refill(ra, bufa, lda)
        _scatters(ra, bufa, sta)
        _wait_refill(rb, bufb, ldb)
        _scatters(rb, bufb, stb)
        _wait_scatters(ra, bufa, sta)

        @pl.when(q <= nrounds // 2 - 2)
        def _():
            _refill(ra + 2, bufa, lda)

        return carry

    lax.fori_loop(0, nrounds // 2, body, 0)
    _wait_scatters(nrounds - 1, bufb, stb)


def _gather_phase(out, table, gidx, bufa, bufb, ga, gb, oa, ob, wid):
    ebase = wid * EPW

    def _gathers(r, buf, sem):
        for k in range(NSTR):
            pltpu.async_copy(table.at[gidx.at[r * NSTR + k]],
                             buf.at[pl.ds(k * IDXW, IDXW)], sem)

    def _wait_gathers(r, buf, sem):
        for k in range(NSTR):
            pltpu.make_async_copy(table.at[gidx.at[r * NSTR + k]],
                                  buf.at[pl.ds(k * IDXW, IDXW)], sem).wait()

    def _out(r, buf, sem):
        pltpu.async_copy(buf, out.at[pl.ds(ebase + r * CHUNK, CHUNK)], sem)

    def _wait_out(r, buf, sem):
        pltpu.make_async_copy(
            buf, out.at[pl.ds(ebase + r * CHUNK, CHUNK)], sem).wait()

    _gathers(0, bufa, ga)

    def body(q, carry):
        ra = 2 * q
        rb = 2 * q + 1

        @pl.when(q >= 1)
        def _():
            _wait_gathers(rb - 2, bufb, gb)
            _out(rb - 2, bufb, ob)

        _wait_gathers(ra, bufa, ga)
        _out(ra, bufa, oa)

        @pl.when(q >= 1)
        def _():
            _wait_out(rb - 2, bufb, ob)

        _gathers(rb, bufb, gb)
        _wait_out(ra, bufa, oa)

        @pl.when(q <= GROUNDS // 2 - 2)
        def _():
            _gathers(ra + 2, bufa, ga)

        return carry

    lax.fori_loop(0, GROUNDS // 2, body, 0)
    _wait_gathers(GROUNDS - 1, bufb, gb)
    _out(GROUNDS - 1, bufb, ob)
    _wait_out(GROUNDS - 1, bufb, ob)


@functools.partial(
    pl.kernel,
    out_type=jax.ShapeDtypeStruct((E, L), jnp.float32),
    mesh=_mesh(),
    compiler_params=_sc_params,
    scratch_types=[
        pltpu.VMEM((EPW // IDXW, IDXW), jnp.int32),
        pltpu.VMEM((N // IDXW, IDXW), jnp.int32),
        pltpu.VMEM((CHUNK, L), jnp.float32),
        pltpu.VMEM((CHUNK, L), jnp.float32),
        pltpu.VMEM_SHARED((N, L), jnp.float32),
    ] + [pltpu.SemaphoreType.DMA] * 4,
)
def _sc_gather2(parts, eidx3, iota2d, out,
                gidx, idt, bufa, bufb, table, s0, s1, s2, s3):
    cid = lax.axis_index("c")
    sid = lax.axis_index("s")
    wid = cid * NS + sid
    rows = N // NS
    pltpu.sync_copy(parts.at[pl.ds(sid * rows, rows)],
                    table.at[pl.ds(sid * rows, rows)])
    pltpu.sync_copy(
        eidx3.at[0, pl.ds(wid * (EPW // IDXW), EPW // IDXW)], gidx)
    pltpu.sync_copy(iota2d, idt)
    ncomb = rows // IDXW
    for c in range(ncomb):
        r0 = N + sid * rows + c * IDXW
        pltpu.sync_copy(parts.at[pl.ds(r0, IDXW)], bufa.at[pl.ds(0, IDXW)])
        pltpu.sync_copy(bufa.at[pl.ds(0, IDXW)],
                        table.at[idt.at[sid * ncomb + c]], add=True)
    plsc.subcore_barrier()
    _gather_phase(out, table, gidx, bufa, bufb, s0, s1, s2, s3, wid)


@functools.partial(
    pl.kernel,
    out_type=jax.ShapeDtypeStruct((E, L), jnp.float32),
    mesh=_mesh(),
    compiler_params=_sc_params,
    scratch_types=[
        pltpu.VMEM((EPW // IDXW, IDXW), jnp.int32),
        pltpu.VMEM((CHUNK, L), jnp.float32),
        pltpu.VMEM((CHUNK, L), jnp.float32),
        pltpu.VMEM_SHARED((N, L), jnp.float32),
    ] + [pltpu.SemaphoreType.DMA] * 4,
)
def _sc_gather0(tab_hbm, eidx3, out, gidx, bufa, bufb, table,
                s0, s1, s2, s3):
    cid = lax.axis_index("c")
    sid = lax.axis_index("s")
    wid = cid * NS + sid
    pltpu.sync_copy(tab_hbm.at[pl.ds(sid * ROWS_MAIN, ROWS_MAIN)],
                    table.at[pl.ds(sid * ROWS_MAIN, ROWS_MAIN)])

    @pl.when(sid == 0)
    def _():
        pltpu.sync_copy(tab_hbm.at[pl.ds(TAIL_BASE, TAIL)],
                        table.at[pl.ds(TAIL_BASE, TAIL)])

    pltpu.sync_copy(
        eidx3.at[0, pl.ds(wid * (EPW // IDXW), EPW // IDXW)], gidx)
    plsc.subcore_barrier()
    _gather_phase(out, table, gidx, bufa, bufb, s0, s1, s2, s3, wid)


@functools.partial(
    pl.kernel,
    out_type=jax.ShapeDtypeStruct((NC * N, L), jnp.float32),
    mesh=_mesh(),
    compiler_params=_sc_params,
    scratch_types=[
        pltpu.VMEM((EPW // IDXW, IDXW), jnp.int32),
        pltpu.VMEM((CHUNK, L), jnp.float32),
        pltpu.VMEM((CHUNK, L), jnp.float32),
        pltpu.VMEM_SHARED((N, L), jnp.float32),
    ] + [pltpu.SemaphoreType.DMA] * 4,
)
def _sc_segsum(msgs, eidx3, ztab, out,
               didx, bufa, bufb, table, s0, s1, s2, s3):
    cid = lax.axis_index("c")
    sid = lax.axis_index("s")
    wid = cid * NS + sid
    _zero_table(ztab, table, sid)
    pltpu.sync_copy(
        eidx3.at[1, pl.ds(wid * (EPW // IDXW), EPW // IDXW)], didx)
    plsc.subcore_barrier()
    _scatter_phase(msgs, table, didx, bufa, bufb, s0, s1, s2, s3,
                   wid * EPW, GROUNDS)
    plsc.subcore_barrier()
    pltpu.sync_copy(table.at[pl.ds(sid * ROWS_MAIN, ROWS_MAIN)],
                    out.at[pl.ds(cid * N + sid * ROWS_MAIN, ROWS_MAIN)])

    @pl.when(sid == 0)
    def _():
        pltpu.sync_copy(table.at[pl.ds(TAIL_BASE, TAIL)],
                        out.at[pl.ds(cid * N + TAIL_BASE, TAIL)])


E2 = E // 2
N2 = N // 2
_NBN = 5
_BN = N2 // _NBN
_NBE = 40
_BE = E2 // _NBE


def _block_diag2(w):
    k, m = w.shape
    z = jnp.zeros((k, m), w.dtype)
    return jnp.concatenate([
        jnp.concatenate([w, z], axis=1),
        jnp.concatenate([z, w], axis=1),
    ], axis=0)


def _prep_body(nf, w, b, o):
    o[...] = jnp.dot(nf[...], w[...],
                     preferred_element_type=jnp.float32) + b[...]


def _tc_prep(node_feat2, w2, b2):
    return pl.pallas_call(
        _prep_body,
        grid=(_NBN,),
        in_specs=[
            pl.BlockSpec((_BN, 2 * DF), lambda i: (i, 0)),
            pl.BlockSpec((2 * DF, 2 * L), lambda i: (0, 0)),
            pl.BlockSpec((1, 2 * L), lambda i: (0, 0)),
        ],
        out_specs=pl.BlockSpec((_BN, 2 * L), lambda i: (i, 0)),
        out_shape=jax.ShapeDtypeStruct((N2, 2 * L), jnp.float32),
    )(node_feat2, w2, b2)


def _input_body(ef, g0, w, b, msg, cur):
    m = jnp.dot(ef[...], w[...],
                preferred_element_type=jnp.float32) + b[...] + g0[...]
    msg[...] = m
    cur[...] = jnp.maximum(m, 0.0)


def _tc_input(edge_feat2, gath0, w2, b2):
    return pl.pallas_call(
        _input_body,
        grid=(_NBE,),
        in_specs=[
            pl.BlockSpec((_BE, 2 * DE), lambda i: (i, 0)),
            pl.BlockSpec((_BE, 2 * L), lambda i: (i, 0)),
            pl.BlockSpec((2 * DE, 2 * L), lambda i: (0, 0)),
            pl.BlockSpec((1, 2 * L), lambda i: (0, 0)),
        ],
        out_specs=[
            pl.BlockSpec((_BE, 2 * L), lambda i: (i, 0)),
            pl.BlockSpec((_BE, 2 * L), lambda i: (i, 0)),
        ],
        out_shape=[
            jax.ShapeDtypeStruct((E2, 2 * L), jnp.float32),
            jax.ShapeDtypeStruct((E2, 2 * L), jnp.float32),
        ],
    )(edge_feat2, gath0, w2, b2)


def _conv_body(g, cr, m, w, b, o):
    e2e = g[...] - cr[...]
    o[...] = jnp.maximum(
        jnp.dot(e2e, w[...], preferred_element_type=jnp.float32)
        + b[...] + m[...], 0.0)


def _tc_conv(gath, cur, msg, w2, b2):
    return pl.pallas_call(
        _conv_body,
        grid=(_NBE,),
        in_specs=[
            pl.BlockSpec((_BE, 2 * L), lambda i: (i, 0)),
            pl.BlockSpec((_BE, 2 * L), lambda i: ((i + _NBE // 2) % _NBE, 0)),
            pl.BlockSpec((_BE, 2 * L), lambda i: (i, 0)),
            pl.BlockSpec((2 * L, 2 * L), lambda i: (0, 0)),
            pl.BlockSpec((1, 2 * L), lambda i: (0, 0)),
        ],
        out_specs=pl.BlockSpec((_BE, 2 * L), lambda i: (i, 0)),
        out_shape=jax.ShapeDtypeStruct((E2, 2 * L), jnp.float32),
    )(gath, cur, msg, w2, b2)


def _out_body(tab, tabb, gid, w, b, y):
    i = pl.program_id(0)
    h = jnp.maximum(tab[...] + tabb[...], 0.0)
    act = jnp.maximum(
        jnp.dot(h, w[...], preferred_element_type=jnp.float32) + b[...], 0.0)
    ids = gid[...]
    iota = lax.broadcasted_iota(jnp.int32, (1, G), 1)
    oh_even = (ids[:, 0:1] == iota).astype(jnp.float32)
    oh_odd = (ids[:, 1:2] == iota).astype(jnp.float32)
    contrib = (
        lax.dot_general(oh_even, act[:, :OUT], (((0,), (0,)), ((), ())),
                        preferred_element_type=jnp.float32)
        + lax.dot_general(oh_odd, act[:, OUT:], (((0,), (0,)), ((), ())),
                          preferred_element_type=jnp.float32))

    @pl.when(i == 0)
    def _():
        y[...] = jnp.zeros_like(y)

    y[...] += contrib

    @pl.when(i == _NBN - 1)
    def _():
        y[...] = jnp.maximum(y[...], 0.0)


def _tc_out(tab2, tab2b, gid2, w2, b2):
    return pl.pallas_call(
        _out_body,
        grid=(_NBN,),
        in_specs=[
            pl.BlockSpec((_BN, 2 * L), lambda i: (i, 0)),
            pl.BlockSpec((_BN, 2 * L), lambda i: (_NBN + i, 0)),
            pl.BlockSpec((_BN, 2), lambda i: (i, 0)),
            pl.BlockSpec((2 * L, 2 * OUT), lambda i: (0, 0)),
            pl.BlockSpec((1, 2 * OUT), lambda i: (0, 0)),
        ],
        out_specs=pl.BlockSpec((G, OUT), lambda i: (0, 0)),
        out_shape=jax.ShapeDtypeStruct((G, OUT), jnp.float32),
    )(tab2, tab2b, gid2, w2, b2)


def kernel(node_feat, edge_feat, edge_index, graph_ids,
           W_n2l, b_n2l, W_e2l, b_e2l, W_conv, b_conv, W_out, b_out):
    eidx3 = edge_index.reshape(2, E // IDXW, IDXW)
    ztab = jnp.zeros((N, L), jnp.float32)
    w2_n2l = _block_diag2(W_n2l)
    w2_e2l = _block_diag2(W_e2l)
    w2_conv = _block_diag2(W_conv)
    w2_out = _block_diag2(W_out)
    b2_n2l = jnp.concatenate([b_n2l, b_n2l]).reshape(1, 2 * L)
    b2_e2l = jnp.concatenate([b_e2l, b_e2l]).reshape(1, 2 * L)
    b2_conv = jnp.concatenate([b_conv, b_conv]).reshape(1, 2 * L)
    b2_out = jnp.concatenate([b_out, b_out]).reshape(1, 2 * OUT)

    node_lin2 = _tc_prep(node_feat.reshape(N2, 2 * DF), w2_n2l, b2_n2l)
    gath0 = _sc_gather0(node_lin2.reshape(N, L), eidx3)
    input_msg, cur = _tc_input(edge_feat.reshape(E2, 2 * DE),
                               gath0.reshape(E2, 2 * L), w2_e2l, b2_e2l)
    iota2d = jnp.arange(N, dtype=jnp.int32).reshape(N // IDXW, IDXW)
    for _ in range(3):
        parts = _sc_segsum(cur.reshape(E, L), eidx3, ztab)
        gath = _sc_gather2(parts, eidx3, iota2d)
        cur = _tc_conv(gath.reshape(E2, 2 * L), cur, input_msg,
                       w2_conv, b2_conv)
    tab = _sc_segsum(cur.reshape(E, L), eidx3, ztab)
    tabf = tab.reshape(N, 2 * L)
    return _tc_out(tabf, tabf, graph_ids.reshape(N2, 2), w2_out, b2_out)


# speedup vs baseline: 1.0825x; 1.0825x over previous
import functools

import jax
import jax.numpy as jnp
from jax import lax
from jax.experimental import pallas as pl
from jax.experimental.pallas import tpu as pltpu
from jax.experimental.pallas import tpu_sc as plsc

N = 10000
E = 320000
DF = 128
DE = 16
L = 64
OUT = 64
G = 64

NC = 2
NS = 16
NW = NC * NS

IDXW = 125
CHUNK = 250
SPT = E // NS
SROUNDS = SPT // CHUNK
EPW = E // NW
GROUNDS = EPW // CHUNK
NSTR = CHUNK // IDXW

ROWS_MAIN = 624
TAIL_BASE = NS * ROWS_MAIN
TAIL = N - TAIL_BASE
HALF = N // NC
WOUT = 312
WTAIL_BASE = NS * WOUT
WTAIL = HALF - WTAIL_BASE

_mesh = functools.partial(
    plsc.VectorSubcoreMesh, core_axis_name="c", subcore_axis_name="s",
    num_cores=NC, num_subcores=NS)

_sc_params = pltpu.CompilerParams(use_tc_tiling_on_sc=False)


def _zero_table(ztab, table, sid):
    pltpu.sync_copy(ztab.at[pl.ds(sid * ROWS_MAIN, ROWS_MAIN)],
                    table.at[pl.ds(sid * ROWS_MAIN, ROWS_MAIN)])

    @pl.when(sid == 0)
    def _():
        pltpu.sync_copy(ztab.at[pl.ds(TAIL_BASE, TAIL)],
                        table.at[pl.ds(TAIL_BASE, TAIL)])


def _scatter_phase(msgs, table, didx, bufa, bufb, lda, ldb, sta, stb,
                   ebase, nrounds):
    def _refill(r, buf, sem):
        pltpu.async_copy(msgs.at[pl.ds(ebase + r * CHUNK, CHUNK)], buf, sem)

    def _wait_refill(r, buf, sem):
        pltpu.make_async_copy(
            msgs.at[pl.ds(ebase + r * CHUNK, CHUNK)], buf, sem).wait()

    def _scatters(r, buf, sem):
        for k in range(NSTR):
            pltpu.async_copy(buf.at[pl.ds(k * IDXW, IDXW)],
                             table.at[didx.at[r * NSTR + k]], sem, add=True)

    def _wait_scatters(r, buf, sem):
        for k in range(NSTR):
            pltpu.make_async_copy(buf.at[pl.ds(k * IDXW, IDXW)],
                                  table.at[didx.at[r * NSTR + k]], sem).wait()

    _refill(0, bufa, lda)

    def body(q, carry):
        ra = 2 * q
        rb = 2 * q + 1

        @pl.when(q >= 1)
        def _():
            _wait_scatters(rb - 2, bufb, stb)

        _refill(rb, bufb, ldb)
        _wait_refill(ra, bufa, lda)
        _scatters(ra, bufa, sta)
        _wait_scatters(ra, bufa, sta)

        @pl.when(q <= nrounds // 2 - 2)
        def _():
            _refill(ra + 2, bufa, lda)

        _wait_refill(rb, bufb, ldb)
        _scatters(rb, bufb, stb)
        return carry

    lax.fori_loop(0, nrounds // 2, body, 0)
    _wait_scatters(nrounds - 1, bufb, stb)


def _gather_phase(out, table, gidx, bufa, bufb, ga, gb, oa, ob, wid):
    ebase = wid * EPW

    def _gathers(r, buf, sem):
        for k in range(NSTR):
            pltpu.async_copy(table.at[gidx.at[r * NSTR + k]],
                             buf.at[pl.ds(k * IDXW, IDXW)], sem)

    def _wait_gathers(r, buf, sem):
        for k in range(NSTR):
            pltpu.make_async_copy(table.at[gidx.at[r * NSTR + k]],
                                  buf.at[pl.ds(k * IDXW, IDXW)], sem).wait()

    def _out(r, buf, sem):
        pltpu.async_copy(buf, out.at[pl.ds(ebase + r * CHUNK, CHUNK)], sem)

    def _wait_out(r, buf, sem):
        pltpu.make_async_copy(
            buf, out.at[pl.ds(ebase + r * CHUNK, CHUNK)], sem).wait()

    _gathers(0, bufa, ga)

    def body(q, carry):
        ra = 2 * q
        rb = 2 * q + 1

        @pl.when(q >= 1)
        def _():
            _wait_gathers(rb - 2, bufb, gb)
            _out(rb - 2, bufb, ob)

        _wait_gathers(ra, bufa, ga)
        _out(ra, bufa, oa)

        @pl.when(q >= 1)
        def _():
            _wait_out(rb - 2, bufb, ob)

        _gathers(rb, bufb, gb)
        _wait_out(ra, bufa, oa)

        @pl.when(q <= GROUNDS // 2 - 2)
        def _():
            _gathers(ra + 2, bufa, ga)

        return carry

    lax.fori_loop(0, GROUNDS // 2, body, 0)
    _wait_gathers(GROUNDS - 1, bufb, gb)
    _out(GROUNDS - 1, bufb, ob)
    _wait_out(GROUNDS - 1, bufb, ob)


@functools.partial(
    pl.kernel,
    out_type=jax.ShapeDtypeStruct((E, L), jnp.float32),
    mesh=_mesh(),
    compiler_params=_sc_params,
    scratch_types=[
        pltpu.VMEM((EPW // IDXW, IDXW), jnp.int32),
        pltpu.VMEM((N // IDXW, IDXW), jnp.int32),
        pltpu.VMEM((CHUNK, L), jnp.float32),
        pltpu.VMEM((CHUNK, L), jnp.float32),
        pltpu.VMEM_SHARED((N, L), jnp.float32),
    ] + [pltpu.SemaphoreType.DMA] * 4,
)
def _sc_gather2(parts, eidx3, iota2d, out,
                gidx, idt, bufa, bufb, table, s0, s1, s2, s3):
    cid = lax.axis_index("c")
    sid = lax.axis_index("s")
    wid = cid * NS + sid
    rows = N // NS
    pltpu.sync_copy(parts.at[pl.ds(sid * rows, rows)],
                    table.at[pl.ds(sid * rows, rows)])
    pltpu.sync_copy(
        eidx3.at[0, pl.ds(wid * (EPW // IDXW), EPW // IDXW)], gidx)
    pltpu.sync_copy(iota2d, idt)
    ncomb = rows // IDXW
    for c in range(ncomb):
        r0 = N + sid * rows + c * IDXW
        pltpu.sync_copy(parts.at[pl.ds(r0, IDXW)], bufa.at[pl.ds(0, IDXW)])
        pltpu.sync_copy(bufa.at[pl.ds(0, IDXW)],
                        table.at[idt.at[sid * ncomb + c]], add=True)
    plsc.subcore_barrier()
    _gather_phase(out, table, gidx, bufa, bufb, s0, s1, s2, s3, wid)


@functools.partial(
    pl.kernel,
    out_type=jax.ShapeDtypeStruct((E, L), jnp.float32),
    mesh=_mesh(),
    compiler_params=_sc_params,
    scratch_types=[
        pltpu.VMEM((EPW // IDXW, IDXW), jnp.int32),
        pltpu.VMEM((CHUNK, L), jnp.float32),
        pltpu.VMEM((CHUNK, L), jnp.float32),
        pltpu.VMEM_SHARED((N, L), jnp.float32),
    ] + [pltpu.SemaphoreType.DMA] * 4,
)
def _sc_gather0(tab_hbm, eidx3, out, gidx, bufa, bufb, table,
                s0, s1, s2, s3):
    cid = lax.axis_index("c")
    sid = lax.axis_index("s")
    wid = cid * NS + sid
    pltpu.sync_copy(tab_hbm.at[pl.ds(sid * ROWS_MAIN, ROWS_MAIN)],
                    table.at[pl.ds(sid * ROWS_MAIN, ROWS_MAIN)])

    @pl.when(sid == 0)
    def _():
        pltpu.sync_copy(tab_hbm.at[pl.ds(TAIL_BASE, TAIL)],
                        table.at[pl.ds(TAIL_BASE, TAIL)])

    pltpu.sync_copy(
        eidx3.at[0, pl.ds(wid * (EPW // IDXW), EPW // IDXW)], gidx)
    plsc.subcore_barrier()
    _gather_phase(out, table, gidx, bufa, bufb, s0, s1, s2, s3, wid)


@functools.partial(
    pl.kernel,
    out_type=jax.ShapeDtypeStruct((NC * N, L), jnp.float32),
    mesh=_mesh(),
    compiler_params=_sc_params,
    scratch_types=[
        pltpu.VMEM((EPW // IDXW, IDXW), jnp.int32),
        pltpu.VMEM((CHUNK, L), jnp.float32),
        pltpu.VMEM((CHUNK, L), jnp.float32),
        pltpu.VMEM_SHARED((N, L), jnp.float32),
    ] + [pltpu.SemaphoreType.DMA] * 4,
)
def _sc_segsum(msgs, eidx3, ztab, out,
               didx, bufa, bufb, table, s0, s1, s2, s3):
    cid = lax.axis_index("c")
    sid = lax.axis_index("s")
    wid = cid * NS + sid
    _zero_table(ztab, table, sid)
    pltpu.sync_copy(
        eidx3.at[1, pl.ds(wid * (EPW // IDXW), EPW // IDXW)], didx)
    plsc.subcore_barrier()
    _scatter_phase(msgs, table, didx, bufa, bufb, s0, s1, s2, s3,
                   wid * EPW, GROUNDS)
    plsc.subcore_barrier()
    pltpu.sync_copy(table.at[pl.ds(sid * ROWS_MAIN, ROWS_MAIN)],
                    out.at[pl.ds(cid * N + sid * ROWS_MAIN, ROWS_MAIN)])

    @pl.when(sid == 0)
    def _():
        pltpu.sync_copy(table.at[pl.ds(TAIL_BASE, TAIL)],
                        out.at[pl.ds(cid * N + TAIL_BASE, TAIL)])


E2 = E // 2
N2 = N // 2
_NBN = 5
_BN = N2 // _NBN
_NBE = 40
_BE = E2 // _NBE


def _block_diag2(w):
    k, m = w.shape
    z = jnp.zeros((k, m), w.dtype)
    return jnp.concatenate([
        jnp.concatenate([w, z], axis=1),
        jnp.concatenate([z, w], axis=1),
    ], axis=0)


def _prep_body(nf, w, b, o):
    o[...] = jnp.dot(nf[...], w[...],
                     preferred_element_type=jnp.float32) + b[...]


def _tc_prep(node_feat2, w2, b2):
    return pl.pallas_call(
        _prep_body,
        grid=(_NBN,),
        in_specs=[
            pl.BlockSpec((_BN, 2 * DF), lambda i: (i, 0)),
            pl.BlockSpec((2 * DF, 2 * L), lambda i: (0, 0)),
            pl.BlockSpec((1, 2 * L), lambda i: (0, 0)),
        ],
        out_specs=pl.BlockSpec((_BN, 2 * L), lambda i: (i, 0)),
        out_shape=jax.ShapeDtypeStruct((N2, 2 * L), jnp.float32),
    )(node_feat2, w2, b2)


def _input_body(ef, g0, w, b, msg, cur):
    m = jnp.dot(ef[...], w[...],
                preferred_element_type=jnp.float32) + b[...] + g0[...]
    msg[...] = m
    cur[...] = jnp.maximum(m, 0.0)


def _tc_input(edge_feat2, gath0, w2, b2):
    return pl.pallas_call(
        _input_body,
        grid=(_NBE,),
        in_specs=[
            pl.BlockSpec((_BE, 2 * DE), lambda i: (i, 0)),
            pl.BlockSpec((_BE, 2 * L), lambda i: (i, 0)),
            pl.BlockSpec((2 * DE, 2 * L), lambda i: (0, 0)),
            pl.BlockSpec((1, 2 * L), lambda i: (0, 0)),
        ],
        out_specs=[
            pl.BlockSpec((_BE, 2 * L), lambda i: (i, 0)),
            pl.BlockSpec((_BE, 2 * L), lambda i: (i, 0)),
        ],
        out_shape=[
            jax.ShapeDtypeStruct((E2, 2 * L), jnp.float32),
            jax.ShapeDtypeStruct((E2, 2 * L), jnp.float32),
        ],
    )(edge_feat2, gath0, w2, b2)


def _conv_body(g, cr, m, w, b, o):
    e2e = g[...] - cr[...]
    o[...] = jnp.maximum(
        jnp.dot(e2e, w[...], preferred_element_type=jnp.float32)
        + b[...] + m[...], 0.0)


def _tc_conv(gath, cur, msg, w2, b2):
    return pl.pallas_call(
        _conv_body,
        grid=(_NBE,),
        in_specs=[
            pl.BlockSpec((_BE, 2 * L), lambda i: (i, 0)),
            pl.BlockSpec((_BE, 2 * L), lambda i: ((i + _NBE // 2) % _NBE, 0)),
            pl.BlockSpec((_BE, 2 * L), lambda i: (i, 0)),
            pl.BlockSpec((2 * L, 2 * L), lambda i: (0, 0)),
            pl.BlockSpec((1, 2 * L), lambda i: (0, 0)),
        ],
        out_specs=pl.BlockSpec((_BE, 2 * L), lambda i: (i, 0)),
        out_shape=jax.ShapeDtypeStruct((E2, 2 * L), jnp.float32),
    )(gath, cur, msg, w2, b2)


def _out_body(tab, tabb, gid, w, b, y):
    i = pl.program_id(0)
    h = jnp.maximum(tab[...] + tabb[...], 0.0)
    act = jnp.maximum(
        jnp.dot(h, w[...], preferred_element_type=jnp.float32) + b[...], 0.0)
    ids = gid[...]
    iota = lax.broadcasted_iota(jnp.int32, (1, G), 1)
    oh_even = (ids[:, 0:1] == iota).astype(jnp.float32)
    oh_odd = (ids[:, 1:2] == iota).astype(jnp.float32)
    contrib = (
        lax.dot_general(oh_even, act[:, :OUT], (((0,), (0,)), ((), ())),
                        preferred_element_type=jnp.float32)
        + lax.dot_general(oh_odd, act[:, OUT:], (((0,), (0,)), ((), ())),
                          preferred_element_type=jnp.float32))

    @pl.when(i == 0)
    def _():
        y[...] = jnp.zeros_like(y)

    y[...] += contrib

    @pl.when(i == _NBN - 1)
    def _():
        y[...] = jnp.maximum(y[...], 0.0)


def _tc_out(tab2, tab2b, gid2, w2, b2):
    return pl.pallas_call(
        _out_body,
        grid=(_NBN,),
        in_specs=[
            pl.BlockSpec((_BN, 2 * L), lambda i: (i, 0)),
            pl.BlockSpec((_BN, 2 * L), lambda i: (_NBN + i, 0)),
            pl.BlockSpec((_BN, 2), lambda i: (i, 0)),
            pl.BlockSpec((2 * L, 2 * OUT), lambda i: (0, 0)),
            pl.BlockSpec((1, 2 * OUT), lambda i: (0, 0)),
        ],
        out_specs=pl.BlockSpec((G, OUT), lambda i: (0, 0)),
        out_shape=jax.ShapeDtypeStruct((G, OUT), jnp.float32),
    )(tab2, tab2b, gid2, w2, b2)


def kernel(node_feat, edge_feat, edge_index, graph_ids,
           W_n2l, b_n2l, W_e2l, b_e2l, W_conv, b_conv, W_out, b_out):
    eidx3 = edge_index.reshape(2, E // IDXW, IDXW)
    ztab = jnp.zeros((N, L), jnp.float32)
    w2_n2l = _block_diag2(W_n2l)
    w2_e2l = _block_diag2(W_e2l)
    w2_conv = _block_diag2(W_conv)
    w2_out = _block_diag2(W_out)
    b2_n2l = jnp.concatenate([b_n2l, b_n2l]).reshape(1, 2 * L)
    b2_e2l = jnp.concatenate([b_e2l, b_e2l]).reshape(1, 2 * L)
    b2_conv = jnp.concatenate([b_conv, b_conv]).reshape(1, 2 * L)
    b2_out = jnp.concatenate([b_out, b_out]).reshape(1, 2 * OUT)

    node_lin2 = _tc_prep(node_feat.reshape(N2, 2 * DF), w2_n2l, b2_n2l)
    gath0 = _sc_gather0(node_lin2.reshape(N, L), eidx3)
    input_msg, cur = _tc_input(edge_feat.reshape(E2, 2 * DE),
                               gath0.reshape(E2, 2 * L), w2_e2l, b2_e2l)
    iota2d = jnp.arange(N, dtype=jnp.int32).reshape(N // IDXW, IDXW)
    for _ in range(3):
        parts = _sc_segsum(cur.reshape(E, L), eidx3, ztab)
        gath = _sc_gather2(parts, eidx3, iota2d)
        cur = _tc_conv(gath.reshape(E2, 2 * L), cur, input_msg,
                       w2_conv, b2_conv)
    tab = _sc_segsum(cur.reshape(E, L), eidx3, ztab)
    tabf = tab.reshape(N, 2 * L)
    return _tc_out(tabf, tabf, graph_ids.reshape(N2, 2), w2_out, b2_out)
